# baseline (speedup 1.0000x reference)
import functools

import jax
import jax.numpy as jnp
from jax import lax
from jax.experimental import pallas as pl
from jax.experimental.pallas import tpu as pltpu
from jax.experimental.pallas import tpu_sc as plsc

BATCH = 16384
FEAT = 16

_DN_T = (((1,), (1,)), ((), ()))


def _sc_gather(table, idx):
    info = plsc.get_sparse_core_info()
    nw = info.num_cores * info.num_subcores
    b_per_w = BATCH // nw
    mesh = plsc.VectorSubcoreMesh(core_axis_name="c", subcore_axis_name="s")

    @functools.partial(
        pl.kernel,
        mesh=mesh,
        compiler_params=pltpu.CompilerParams(use_tc_tiling_on_sc=False),
        out_type=jax.ShapeDtypeStruct((BATCH, FEAT), jnp.float32),
        scratch_types=[
            pltpu.VMEM((b_per_w,), jnp.int32),
            pltpu.VMEM((b_per_w, FEAT), jnp.float32),
            pltpu.SemaphoreType.DMA,
        ],
    )
    def gather_k(table_hbm, idx_hbm, out_hbm, idx_v, rows_v, sem):
        wid = lax.axis_index("s") * info.num_cores + lax.axis_index("c")
        base = wid * b_per_w
        pltpu.sync_copy(idx_hbm.at[pl.ds(base, b_per_w)], idx_v)
        pltpu.async_copy(table_hbm.at[idx_v], rows_v, sem).wait()
        pltpu.sync_copy(rows_v, out_hbm.at[pl.ds(base, b_per_w)])

    return gather_k(table, idx)


def _tc_mlp(xg, ig, w1, b1, w2, b2, w3, b3):
    blk = 2048
    grid = BATCH // blk

    def body(x_ref, ig_ref, w1_ref, b1_ref, w2_ref, b2_ref, w3_ref, b3_ref,
             o_ref):
        x = x_ref[:]
        g = ig_ref[:]
        w1 = w1_ref[:]
        h = lax.dot_general(x[:, 2:10], w1[:, 0:8], _DN_T,
                            preferred_element_type=jnp.float32)
        h = h + lax.dot_general(g, w1[:, 8:9], _DN_T,
                                preferred_element_type=jnp.float32)
        h = jnp.maximum(h + b1_ref[:], 0.0)
        h = lax.dot_general(h, w2_ref[:], _DN_T,
                            preferred_element_type=jnp.float32)
        h = jnp.maximum(h + b2_ref[:], 0.0)
        r = lax.dot_general(h, w3_ref[:], _DN_T,
                            preferred_element_type=jnp.float32)
        o_ref[:] = x[:, 0:2] + r + b3_ref[:]

    full = lambda shape: pl.BlockSpec(shape, lambda i: (0, 0))
    return pl.pallas_call(
        body,
        grid=(grid,),
        in_specs=[
            pl.BlockSpec((blk, FEAT), lambda i: (i, 0)),
            pl.BlockSpec((blk, 1), lambda i: (i, 0)),
            full((32, 9)),
            full((1, 32)),
            full((16, 32)),
            full((1, 16)),
            full((2, 16)),
            full((1, 2)),
        ],
        out_specs=pl.BlockSpec((blk, 2), lambda i: (i, 0)),
        out_shape=jax.ShapeDtypeStruct((BATCH, 2), jnp.float32),
    )(xg, ig, w1, b1, w2, b2, w3, b3)


def kernel(action_idx, is_ground, physics_params, action_emb, W1, b1, W2, b2,
           W3, b3, gravity):
    idx = action_idx.astype(jnp.int32)
    n = physics_params.shape[0]
    table = jnp.concatenate(
        [physics_params[:, :2], action_emb,
         jnp.zeros((n, FEAT - 10), jnp.float32)], axis=1)
    xg = _sc_gather(table, idx)
    out = _tc_mlp(xg, is_ground.reshape(BATCH, 1), W1, b1.reshape(1, 32),
                  W2, b2.reshape(1, 16), W3, b3.reshape(1, 2))
    return (out, gravity)


# speedup vs baseline: 1.0512x; 1.0512x over previous
import functools

import jax
import jax.numpy as jnp
from jax import lax
from jax.experimental import pallas as pl
from jax.experimental.pallas import tpu as pltpu
from jax.experimental.pallas import tpu_sc as plsc

BATCH = 16384
FEAT = 16
PACK = 128 // FEAT


def _sc_gather(table, idx):
    info = plsc.get_sparse_core_info()
    nw = info.num_cores * info.num_subcores
    b_per_w = BATCH // nw
    mesh = plsc.VectorSubcoreMesh(core_axis_name="c", subcore_axis_name="s")

    half = b_per_w // 2

    @functools.partial(
        pl.kernel,
        mesh=mesh,
        compiler_params=pltpu.CompilerParams(use_tc_tiling_on_sc=False),
        out_type=jax.ShapeDtypeStruct((BATCH, FEAT), jnp.float32),
        scratch_types=[
            pltpu.VMEM((half,), jnp.int32),
            pltpu.VMEM((half,), jnp.int32),
            pltpu.VMEM((half, FEAT), jnp.float32),
            pltpu.VMEM((half, FEAT), jnp.float32),
            pltpu.SemaphoreType.DMA,
            pltpu.SemaphoreType.DMA,
            pltpu.SemaphoreType.DMA,
            pltpu.SemaphoreType.DMA,
        ],
    )
    def gather_k(table_hbm, idx_hbm, out_hbm, idx_v0, idx_v1, rows_v0,
                 rows_v1, si0, si1, sg0, sg1):
        wid = lax.axis_index("s") * info.num_cores + lax.axis_index("c")
        base = wid * b_per_w
        i0 = pltpu.async_copy(idx_hbm.at[pl.ds(base, half)], idx_v0, si0)
        i1 = pltpu.async_copy(idx_hbm.at[pl.ds(base + half, half)], idx_v1,
                              si1)
        i0.wait()
        g0 = pltpu.async_copy(table_hbm.at[idx_v0], rows_v0, sg0)
        i1.wait()
        g1 = pltpu.async_copy(table_hbm.at[idx_v1], rows_v1, sg1)
        g0.wait()
        w0 = pltpu.async_copy(rows_v0, out_hbm.at[pl.ds(base, half)], si0)
        g1.wait()
        w1 = pltpu.async_copy(rows_v1, out_hbm.at[pl.ds(base + half, half)],
                              si1)
        w0.wait()
        w1.wait()

    return gather_k(table, idx)


def _tc_mlp(xp, igp, w1bd, b1bd, w2bd, b2bd, w3bd, sbd, b3bd):
    rows = BATCH // PACK
    blk = 256
    grid = rows // blk

    def body(x_ref, ig_ref, w1_ref, b1_ref, w2_ref, b2_ref, w3_ref, s_ref,
             b3_ref, o_ref):
        x = x_ref[:] + ig_ref[:]
        h = jnp.dot(x, w1_ref[:], preferred_element_type=jnp.float32)
        h = jnp.maximum(h + b1_ref[:], 0.0)
        h = jnp.dot(h, w2_ref[:], preferred_element_type=jnp.float32)
        h = jnp.maximum(h + b2_ref[:], 0.0)
        r = jnp.dot(h, w3_ref[:], preferred_element_type=jnp.float32)
        base = jnp.dot(x, s_ref[:], preferred_element_type=jnp.float32)
        o_ref[:] = base + r + b3_ref[:]

    full = lambda shape: pl.BlockSpec(shape, lambda i: (0, 0))
    return pl.pallas_call(
        body,
        grid=(grid,),
        in_specs=[
            pl.BlockSpec((blk, 128), lambda i: (i, 0)),
            pl.BlockSpec((blk, 128), lambda i: (i, 0)),
            full((128, 256)),
            full((1, 256)),
            full((256, 128)),
            full((1, 128)),
            full((128, 16)),
            full((128, 16)),
            full((1, 16)),
        ],
        out_specs=pl.BlockSpec((blk, 16), lambda i: (i, 0)),
        out_shape=jax.ShapeDtypeStruct((rows, 16), jnp.float32),
    )(xp, igp, w1bd, b1bd, w2bd, b2bd, w3bd, sbd, b3bd)


def kernel(action_idx, is_ground, physics_params, action_emb, W1, b1, W2, b2,
           W3, b3, gravity):
    idx = action_idx.astype(jnp.int32)
    n = physics_params.shape[0]
    f32 = jnp.float32
    table = jnp.concatenate(
        [physics_params[:, :2], action_emb, jnp.zeros((n, FEAT - 10), f32)],
        axis=1)
    eye8 = jnp.eye(PACK, dtype=f32)
    w1e = (jnp.zeros((FEAT, 32), f32).at[2:10].set(W1[:, :8].T)
           .at[10].set(W1[:, 8]))
    w1bd = jnp.kron(eye8, w1e)
    w2bd = jnp.kron(eye8, W2.T)
    w3bd = jnp.kron(eye8, W3.T)
    sel = jnp.zeros((FEAT, 2), f32).at[0, 0].set(1.0).at[1, 1].set(1.0)
    sbd = jnp.kron(eye8, sel)
    b1bd = jnp.tile(b1, PACK).reshape(1, 256)
    b2bd = jnp.tile(b2, PACK).reshape(1, 128)
    b3bd = jnp.tile(b3, PACK).reshape(1, 16)

    onehot10 = jnp.zeros((1, 1, FEAT), f32).at[0, 0, 10].set(1.0)
    igp = (is_ground.reshape(BATCH // PACK, PACK, 1) * onehot10).reshape(
        BATCH // PACK, 128)

    xg = _sc_gather(table, idx)
    xp = xg.reshape(BATCH // PACK, 128)
    outp = _tc_mlp(xp, igp, w1bd, b1bd, w2bd, b2bd, w3bd, sbd, b3bd)
    return (outp.reshape(BATCH, 2), gravity)


# speedup vs baseline: 1.2759x; 1.2138x over previous
import functools

import jax
import jax.numpy as jnp
from jax import lax
from jax.experimental import pallas as pl
from jax.experimental.pallas import tpu as pltpu
from jax.experimental.pallas import tpu_sc as plsc

BATCH = 16384
FEAT = 16
PACK = 128 // FEAT

_DN_T = (((1,), (1,)), ((), ()))


def _sc_gather(table, idx):
    info = plsc.get_sparse_core_info()
    nw = info.num_cores * info.num_subcores
    b_per_w = BATCH // nw
    mesh = plsc.VectorSubcoreMesh(core_axis_name="c", subcore_axis_name="s")
    half = b_per_w // 2

    @functools.partial(
        pl.kernel,
        mesh=mesh,
        compiler_params=pltpu.CompilerParams(use_tc_tiling_on_sc=False),
        out_type=jax.ShapeDtypeStruct((BATCH, FEAT), jnp.float32),
        scratch_types=[
            pltpu.VMEM((half,), jnp.int32),
            pltpu.VMEM((half,), jnp.int32),
            pltpu.VMEM((half, FEAT), jnp.float32),
            pltpu.VMEM((half, FEAT), jnp.float32),
            pltpu.SemaphoreType.DMA,
            pltpu.SemaphoreType.DMA,
            pltpu.SemaphoreType.DMA,
            pltpu.SemaphoreType.DMA,
        ],
    )
    def gather_k(table_hbm, idx_hbm, out_hbm, idx_v0, idx_v1, rows_v0,
                 rows_v1, si0, si1, sg0, sg1):
        wid = lax.axis_index("s") * info.num_cores + lax.axis_index("c")
        base = wid * b_per_w
        i0 = pltpu.async_copy(idx_hbm.at[pl.ds(base, half)], idx_v0, si0)
        i1 = pltpu.async_copy(idx_hbm.at[pl.ds(base + half, half)], idx_v1,
                              si1)
        i0.wait()
        g0 = pltpu.async_copy(table_hbm.at[idx_v0], rows_v0, sg0)
        i1.wait()
        g1 = pltpu.async_copy(table_hbm.at[idx_v1], rows_v1, sg1)
        g0.wait()
        w0 = pltpu.async_copy(rows_v0, out_hbm.at[pl.ds(base, half)], si0)
        g1.wait()
        w1 = pltpu.async_copy(rows_v1, out_hbm.at[pl.ds(base + half, half)],
                              si1)
        w0.wait()
        w1.wait()

    return gather_k(table, idx)


def _blockdiag(w_tiled, rows, cols, rblk, cblk):
    r = lax.broadcasted_iota(jnp.int32, (rows, cols), 0)
    c = lax.broadcasted_iota(jnp.int32, (rows, cols), 1)
    return jnp.where((r // rblk) == (c // cblk), w_tiled, 0.0)


def _tc_mlp(xp, igp, w1, b1, w2, b2, w3, b3):
    rows = BATCH // PACK
    blk = 512
    grid = rows // blk
    f32 = jnp.float32

    def body(x_ref, ig_ref, w1_ref, b1_ref, w2_ref, b2_ref, w3_ref, b3_ref,
             o_ref):
        w1t = jnp.concatenate(
            [jnp.zeros((32, 2), f32), w1_ref[:], jnp.zeros((32, 5), f32)],
            axis=1)
        w1k = _blockdiag(jnp.tile(w1t, (8, 8)), 256, 128, 32, 16)
        w2k = _blockdiag(jnp.tile(w2_ref[:], (8, 8)), 128, 256, 16, 32)
        w3k = _blockdiag(jnp.tile(w3_ref[:], (8, 8)), 16, 128, 2, 16)
        sr = lax.broadcasted_iota(jnp.int32, (16, 128), 0)
        sc = lax.broadcasted_iota(jnp.int32, (16, 128), 1)
        selk = jnp.where(sc == 16 * (sr // 2) + (sr % 2), 1.0, 0.0)
        b1k = jnp.tile(b1_ref[:], (1, 8))
        b2k = jnp.tile(b2_ref[:], (1, 8))
        b3k = jnp.tile(b3_ref[:], (1, 8))

        x = x_ref[:] + ig_ref[:]
        h = lax.dot_general(x, w1k, _DN_T, preferred_element_type=f32)
        h = jnp.maximum(h + b1k, 0.0)
        h = lax.dot_general(h, w2k, _DN_T, preferred_element_type=f32)
        h = jnp.maximum(h + b2k, 0.0)
        r = lax.dot_general(h, w3k, _DN_T, preferred_element_type=f32)
        base = lax.dot_general(x, selk, _DN_T, preferred_element_type=f32)
        o_ref[:] = base + r + b3k

    full = lambda shape: pl.BlockSpec(shape, lambda i: (0, 0))
    return pl.pallas_call(
        body,
        grid=(grid,),
        in_specs=[
            pl.BlockSpec((blk, 128), lambda i: (i, 0)),
            pl.BlockSpec((blk, 128), lambda i: (i, 0)),
            full((32, 9)),
            full((1, 32)),
            full((16, 32)),
            full((1, 16)),
            full((2, 16)),
            full((1, 2)),
        ],
        out_specs=pl.BlockSpec((blk, 16), lambda i: (i, 0)),
        out_shape=jax.ShapeDtypeStruct((rows, 16), jnp.float32),
    )(xp, igp, w1, b1, w2, b2, w3, b3)


def kernel(action_idx, is_ground, physics_params, action_emb, W1, b1, W2, b2,
           W3, b3, gravity):
    idx = action_idx.astype(jnp.int32)
    n = physics_params.shape[0]
    f32 = jnp.float32
    table = jnp.concatenate(
        [physics_params[:, :2], action_emb, jnp.zeros((n, FEAT - 10), f32)],
        axis=1)
    onehot10 = jnp.zeros((1, 1, FEAT), f32).at[0, 0, 10].set(1.0)
    igp = (is_ground.reshape(BATCH // PACK, PACK, 1) * onehot10).reshape(
        BATCH // PACK, 128)

    xg = _sc_gather(table, idx)
    xp = xg.reshape(BATCH // PACK, 128)
    outp = _tc_mlp(xp, igp, W1, b1.reshape(1, 32), W2, b2.reshape(1, 16),
                   W3, b3.reshape(1, 2))
    return (outp.reshape(BATCH, 2), gravity)


# speedup vs baseline: 1.3974x; 1.0952x over previous
import functools

import jax
import jax.numpy as jnp
from jax import lax
from jax.experimental import pallas as pl
from jax.experimental.pallas import tpu as pltpu
from jax.experimental.pallas import tpu_sc as plsc

BATCH = 16384
FEAT = 16
PACK = 128 // FEAT

_DN_T = (((1,), (1,)), ((), ()))


def _sc_gather(table, idx):
    info = plsc.get_sparse_core_info()
    nw = info.num_cores * info.num_subcores
    b_per_w = BATCH // nw
    mesh = plsc.VectorSubcoreMesh(core_axis_name="c", subcore_axis_name="s")
    half = b_per_w // 2

    @functools.partial(
        pl.kernel,
        mesh=mesh,
        compiler_params=pltpu.CompilerParams(use_tc_tiling_on_sc=False),
        out_type=jax.ShapeDtypeStruct((BATCH, FEAT), jnp.float32),
        scratch_types=[
            pltpu.VMEM((half,), jnp.int32),
            pltpu.VMEM((half,), jnp.int32),
            pltpu.VMEM((half, FEAT), jnp.float32),
            pltpu.VMEM((half, FEAT), jnp.float32),
            pltpu.SemaphoreType.DMA,
            pltpu.SemaphoreType.DMA,
            pltpu.SemaphoreType.DMA,
            pltpu.SemaphoreType.DMA,
        ],
    )
    def gather_k(table_hbm, idx_hbm, out_hbm, idx_v0, idx_v1, rows_v0,
                 rows_v1, si0, si1, sg0, sg1):
        wid = lax.axis_index("s") * info.num_cores + lax.axis_index("c")
        base = wid * b_per_w
        i0 = pltpu.async_copy(idx_hbm.at[pl.ds(base, half)], idx_v0, si0)
        i1 = pltpu.async_copy(idx_hbm.at[pl.ds(base + half, half)], idx_v1,
                              si1)
        i0.wait()
        g0 = pltpu.async_copy(table_hbm.at[idx_v0], rows_v0, sg0)
        i1.wait()
        g1 = pltpu.async_copy(table_hbm.at[idx_v1], rows_v1, sg1)
        g0.wait()
        w0 = pltpu.async_copy(rows_v0, out_hbm.at[pl.ds(base, half)], si0)
        g1.wait()
        w1 = pltpu.async_copy(rows_v1, out_hbm.at[pl.ds(base + half, half)],
                              si1)
        w0.wait()
        w1.wait()

    return gather_k(table, idx)


def _blockdiag(w_tiled, rows, cols, rblk, cblk):
    r = lax.broadcasted_iota(jnp.int32, (rows, cols), 0)
    c = lax.broadcasted_iota(jnp.int32, (rows, cols), 1)
    return jnp.where((r // rblk) == (c // cblk), w_tiled, 0.0)


def _tc_mlp(xp, igp, w1, b1, w2, b2, w3, b3):
    rows = BATCH // PACK
    blk = 512
    grid = rows // blk
    f32 = jnp.float32

    def body(x_ref, ig_ref, w1_ref, b1_ref, w2_ref, b2_ref, w3_ref, b3_ref,
             o_ref):
        w1t = jnp.concatenate(
            [jnp.zeros((32, 2), f32), w1_ref[:], jnp.zeros((32, 5), f32)],
            axis=1)
        w1k = _blockdiag(jnp.tile(w1t, (8, 8)), 256, 128, 32, 16)
        w2k = _blockdiag(jnp.tile(w2_ref[:], (8, 8)), 128, 256, 16, 32)
        w3k = _blockdiag(jnp.tile(w3_ref[:], (8, 8)), 16, 128, 2, 16)
        sr = lax.broadcasted_iota(jnp.int32, (16, 128), 0)
        sc = lax.broadcasted_iota(jnp.int32, (16, 128), 1)
        selk = jnp.where(sc == 16 * (sr // 2) + (sr % 2), 1.0, 0.0)
        b1k = jnp.tile(b1_ref[:], (1, 8))
        b2k = jnp.tile(b2_ref[:], (1, 8))
        b3k = jnp.tile(b3_ref[:], (1, 8))

        x = x_ref[:] + ig_ref[:]
        h = lax.dot_general(x, w1k, _DN_T, preferred_element_type=f32)
        h = jnp.maximum(h + b1k, 0.0)
        h = lax.dot_general(h, w2k, _DN_T, preferred_element_type=f32)
        h = jnp.maximum(h + b2k, 0.0)
        r = lax.dot_general(h, w3k, _DN_T, preferred_element_type=f32)
        base = lax.dot_general(x, selk, _DN_T, preferred_element_type=f32)
        out = base + r + b3k
        o_ref[:] = out.reshape(blk, PACK, 2)

    full = lambda shape: pl.BlockSpec(shape, lambda i: (0, 0))
    return pl.pallas_call(
        body,
        grid=(grid,),
        in_specs=[
            pl.BlockSpec((blk, 128), lambda i: (i, 0)),
            pl.BlockSpec((blk, 128), lambda i: (i, 0)),
            full((32, 9)),
            full((1, 32)),
            full((16, 32)),
            full((1, 16)),
            full((2, 16)),
            full((1, 2)),
        ],
        out_specs=pl.BlockSpec((blk, PACK, 2), lambda i: (i, 0, 0)),
        out_shape=jax.ShapeDtypeStruct((rows, PACK, 2), jnp.float32),
    )(xp, igp, w1, b1, w2, b2, w3, b3)


def kernel(action_idx, is_ground, physics_params, action_emb, W1, b1, W2, b2,
           W3, b3, gravity):
    idx = action_idx.astype(jnp.int32)
    n = physics_params.shape[0]
    f32 = jnp.float32
    table = jnp.concatenate(
        [physics_params[:, :2], action_emb, jnp.zeros((n, FEAT - 10), f32)],
        axis=1)
    onehot10 = jnp.zeros((1, 1, FEAT), f32).at[0, 0, 10].set(1.0)
    igp = (is_ground.reshape(BATCH // PACK, PACK, 1) * onehot10).reshape(
        BATCH // PACK, 128)

    xg = _sc_gather(table, idx)
    xp = xg.reshape(BATCH // PACK, 128)
    outp = _tc_mlp(xp, igp, W1, b1.reshape(1, 32), W2, b2.reshape(1, 16),
                   W3, b3.reshape(1, 2))
    return (outp.reshape(BATCH, 2), gravity)


# speedup vs baseline: 1.4082x; 1.0077x over previous
import functools

import jax
import jax.numpy as jnp
from jax import lax
from jax.experimental import pallas as pl
from jax.experimental.pallas import tpu as pltpu
from jax.experimental.pallas import tpu_sc as plsc

BATCH = 16384
FEAT = 16
PACK = 128 // FEAT

_DN_T = (((1,), (1,)), ((), ()))


def _sc_gather(table, idx):
    info = plsc.get_sparse_core_info()
    nw = info.num_cores * info.num_subcores
    b_per_w = BATCH // nw
    mesh = plsc.VectorSubcoreMesh(core_axis_name="c", subcore_axis_name="s")
    half = b_per_w // 2

    @functools.partial(
        pl.kernel,
        mesh=mesh,
        compiler_params=pltpu.CompilerParams(use_tc_tiling_on_sc=False),
        out_type=jax.ShapeDtypeStruct((BATCH, FEAT), jnp.float32),
        scratch_types=[
            pltpu.VMEM((half,), jnp.int32),
            pltpu.VMEM((half,), jnp.int32),
            pltpu.VMEM((half, FEAT), jnp.float32),
            pltpu.VMEM((half, FEAT), jnp.float32),
            pltpu.SemaphoreType.DMA,
            pltpu.SemaphoreType.DMA,
            pltpu.SemaphoreType.DMA,
            pltpu.SemaphoreType.DMA,
        ],
    )
    def gather_k(table_hbm, idx_hbm, out_hbm, idx_v0, idx_v1, rows_v0,
                 rows_v1, si0, si1, sg0, sg1):
        wid = lax.axis_index("s") * info.num_cores + lax.axis_index("c")
        base = wid * b_per_w
        i0 = pltpu.async_copy(idx_hbm.at[pl.ds(base, half)], idx_v0, si0)
        i1 = pltpu.async_copy(idx_hbm.at[pl.ds(base + half, half)], idx_v1,
                              si1)
        i0.wait()
        g0 = pltpu.async_copy(table_hbm.at[idx_v0], rows_v0, sg0)
        i1.wait()
        g1 = pltpu.async_copy(table_hbm.at[idx_v1], rows_v1, sg1)
        g0.wait()
        w0 = pltpu.async_copy(rows_v0, out_hbm.at[pl.ds(base, half)], si0)
        g1.wait()
        w1 = pltpu.async_copy(rows_v1, out_hbm.at[pl.ds(base + half, half)],
                              si1)
        w0.wait()
        w1.wait()

    return gather_k(table, idx)


def _blockdiag(w_tiled, rows, cols, rblk, cblk):
    r = lax.broadcasted_iota(jnp.int32, (rows, cols), 0)
    c = lax.broadcasted_iota(jnp.int32, (rows, cols), 1)
    return jnp.where((r // rblk) == (c // cblk), w_tiled, 0.0)


def _tc_mlp(xp, igp, w1, b1, w2, b2, w3, b3):
    rows = BATCH // PACK
    blk = 1024
    grid = rows // blk
    f32 = jnp.float32

    def body(x_ref, ig_ref, w1_ref, b1_ref, w2_ref, b2_ref, w3_ref, b3_ref,
             o_ref):
        w1t = jnp.concatenate(
            [jnp.zeros((32, 2), f32), w1_ref[:], jnp.zeros((32, 5), f32)],
            axis=1)
        w1k = _blockdiag(jnp.tile(w1t, (8, 8)), 256, 128, 32, 16)
        w2k = _blockdiag(jnp.tile(w2_ref[:], (8, 8)), 128, 256, 16, 32)
        w3k = _blockdiag(jnp.tile(w3_ref[:], (8, 8)), 16, 128, 2, 16)
        sr = lax.broadcasted_iota(jnp.int32, (16, 128), 0)
        sc = lax.broadcasted_iota(jnp.int32, (16, 128), 1)
        selk = jnp.where(sc == 16 * (sr // 2) + (sr % 2), 1.0, 0.0)
        b1k = jnp.tile(b1_ref[:], (1, 8))
        b2k = jnp.tile(b2_ref[:], (1, 8))
        b3k = jnp.tile(b3_ref[:], (1, 8))

        x = x_ref[:] + ig_ref[:]
        h = lax.dot_general(x, w1k, _DN_T, preferred_element_type=f32)
        h = jnp.maximum(h + b1k, 0.0)
        h = lax.dot_general(h, w2k, _DN_T, preferred_element_type=f32)
        h = jnp.maximum(h + b2k, 0.0)
        r = lax.dot_general(h, w3k, _DN_T, preferred_element_type=f32)
        base = lax.dot_general(x, selk, _DN_T, preferred_element_type=f32)
        out = base + r + b3k
        o_ref[:] = out.reshape(blk, PACK, 2)

    full = lambda shape: pl.BlockSpec(shape, lambda i: (0, 0))
    return pl.pallas_call(
        body,
        grid=(grid,),
        in_specs=[
            pl.BlockSpec((blk, 128), lambda i: (i, 0)),
            pl.BlockSpec((blk, 128), lambda i: (i, 0)),
            full((32, 9)),
            full((1, 32)),
            full((16, 32)),
            full((1, 16)),
            full((2, 16)),
            full((1, 2)),
        ],
        out_specs=pl.BlockSpec((blk, PACK, 2), lambda i: (i, 0, 0)),
        out_shape=jax.ShapeDtypeStruct((rows, PACK, 2), jnp.float32),
    )(xp, igp, w1, b1, w2, b2, w3, b3)


def kernel(action_idx, is_ground, physics_params, action_emb, W1, b1, W2, b2,
           W3, b3, gravity):
    idx = action_idx.astype(jnp.int32)
    n = physics_params.shape[0]
    f32 = jnp.float32
    table = jnp.concatenate(
        [physics_params[:, :2], action_emb, jnp.zeros((n, FEAT - 10), f32)],
        axis=1)
    onehot10 = jnp.zeros((1, 1, FEAT), f32).at[0, 0, 10].set(1.0)
    igp = (is_ground.reshape(BATCH // PACK, PACK, 1) * onehot10).reshape(
        BATCH // PACK, 128)

    xg = _sc_gather(table, idx)
    xp = xg.reshape(BATCH // PACK, 128)
    outp = _tc_mlp(xp, igp, W1, b1.reshape(1, 32), W2, b2.reshape(1, 16),
                   W3, b3.reshape(1, 2))
    return (outp.reshape(BATCH, 2), gravity)
